# baseline (speedup 1.0000x reference)
import functools

import jax
import jax.numpy as jnp
from jax import lax
from jax.experimental import pallas as pl
from jax.experimental.pallas import tpu as pltpu
from jax.experimental.pallas import tpu_sc as plsc

N = 262144
EPS = 1e-7
NW = 32
PA = N // NW
ITERS = PA // 16

TC_EIGHTHS = 6
SC_A0 = N * TC_EIGHTHS // 8
SD = (N - SC_A0) // NW
SD_ITERS = SD // 16

LN2 = 0.6931471805599453
SQRT2 = 1.4142135623730951
_LOG_COEF = (
    -5.4488729807735065e-06,
    0.9998871159844557,
    -0.49911010866955874,
    0.33800562352226765,
    -0.2740800450170098,
    0.17224595127722797,
)


def _ln(q):
    bits = lax.bitcast_convert_type(q, jnp.int32)
    e = (bits >> 23) - 127
    m = lax.bitcast_convert_type((bits & 0x007FFFFF) | 0x3F800000, jnp.float32)
    big = m > SQRT2
    m = jnp.where(big, m * 0.5, m)
    ef = e.astype(jnp.float32) + jnp.where(big, 1.0, 0.0)
    u = m - 1.0
    p = jnp.full_like(q, _LOG_COEF[5])
    for c in _LOG_COEF[4::-1]:
        p = p * u + c
    return p + ef * LN2


def _sc_cls(ts_hbm, os_hbm, td_hbm, od_hbm, out_hbm,
            ts_v, os_v, t2_v, td_v, od_v, acc_v, s0, s1, s2, s3, s4):
    wid = lax.axis_index("s") * 2 + lax.axis_index("c")
    abase = wid * PA
    dbase = SC_A0 + wid * SD
    wbase = dbase * 4

    c0 = pltpu.async_copy(ts_hbm.at[pl.ds(abase, PA)], ts_v, s0)
    c1 = pltpu.async_copy(os_hbm.at[pl.ds(abase, PA)], os_v, s1)
    c2 = pltpu.async_copy(ts_hbm.at[pl.ds(dbase, SD)], t2_v, s2)
    c3 = pltpu.async_copy(td_hbm.at[pl.ds(wbase, SD * 4)], td_v, s3)
    c4 = pltpu.async_copy(od_hbm.at[pl.ds(wbase, SD * 4)], od_v, s4)
    c0.wait()
    c1.wait()

    def one(a0):
        t = ts_v[pl.ds(a0, 16)]
        p = os_v[pl.ds(a0, 16)]
        valid = t >= 0.0
        pos = t > 0.0
        q = jnp.where(pos, p, 1.0 - p)
        bce = -_ln(q)
        return (jnp.where(valid, bce, 0.0),
                jnp.where(valid, 1.0, 0.0),
                jnp.where(pos, 1.0, 0.0))

    def body(i, carry):
        acc_bce, acc_nv, acc_np = carry
        a0 = pl.multiple_of(i * 64, 64)
        b0, v0, p0 = one(a0)
        b1, v1, p1 = one(a0 + 16)
        b2, v2, p2 = one(a0 + 32)
        b3, v3, p3 = one(a0 + 48)
        return (acc_bce + ((b0 + b1) + (b2 + b3)),
                acc_nv + ((v0 + v1) + (v2 + v3)),
                acc_np + ((p0 + p1) + (p2 + p3)))

    z = jnp.zeros((16,), jnp.float32)
    acc_bce, acc_nv, acc_np = lax.fori_loop(0, ITERS // 4, body, (z, z, z))

    c2.wait()
    c3.wait()
    c4.wait()

    def rbody(i, acc_rg):
        a0 = pl.multiple_of(i * 16, 16)
        pos = t2_v[pl.ds(a0, 16)] > 0.0
        v0 = pl.multiple_of((i >> 3) * 512 + (i & 7) * 16, 16)
        sl = jnp.zeros((16,), jnp.float32)
        for c in range(4):
            b = pl.multiple_of(v0 + c * 128, 16)
            d = od_v[pl.ds(b, 16)] - td_v[pl.ds(b, 16)]
            ad = jnp.abs(d)
            m = jnp.minimum(ad, 1.0)
            sl = sl + m * (ad - 0.5 * m)
        return acc_rg + jnp.where(pos, sl, 0.0)

    acc_rg = lax.fori_loop(0, SD_ITERS, rbody, z)

    acc_v[pl.ds(0, 16)] = acc_bce
    acc_v[pl.ds(16, 16)] = acc_nv
    acc_v[pl.ds(32, 16)] = acc_np
    acc_v[pl.ds(48, 16)] = acc_rg
    pltpu.sync_copy(acc_v, out_hbm.at[wid])


_sc_call = functools.partial(
    pl.kernel,
    out_type=jax.ShapeDtypeStruct((NW, 64), jnp.float32),
    mesh=plsc.VectorSubcoreMesh(core_axis_name="c", subcore_axis_name="s"),
    scratch_types=[
        pltpu.VMEM((PA,), jnp.float32),
        pltpu.VMEM((PA,), jnp.float32),
        pltpu.VMEM((SD,), jnp.float32),
        pltpu.VMEM((SD * 4,), jnp.float32),
        pltpu.VMEM((SD * 4,), jnp.float32),
        pltpu.VMEM((64,), jnp.float32),
        pltpu.SemaphoreType.DMA,
        pltpu.SemaphoreType.DMA,
        pltpu.SemaphoreType.DMA,
        pltpu.SemaphoreType.DMA,
        pltpu.SemaphoreType.DMA,
    ],
    compiler_params=pltpu.CompilerParams(needs_layout_passes=False),
)(_sc_cls)


_GRID = TC_EIGHTHS
_RB = (N // 128) // 8
_DRB = 4 * _RB


def _tc_reg_body(ts_hbm, td_hbm, od_hbm, out_ref, ts_b, td_b, od_b, sem):
    def start(k, slot):
        pltpu.make_async_copy(
            ts_hbm.at[pl.ds(k * _RB, _RB), :], ts_b.at[slot], sem.at[slot, 0]
        ).start()
        pltpu.make_async_copy(
            td_hbm.at[pl.ds(k * _DRB, _DRB), :], td_b.at[slot], sem.at[slot, 1]
        ).start()
        pltpu.make_async_copy(
            od_hbm.at[pl.ds(k * _DRB, _DRB), :], od_b.at[slot], sem.at[slot, 2]
        ).start()

    def wait(k, slot):
        pltpu.make_async_copy(
            ts_hbm.at[pl.ds(k * _RB, _RB), :], ts_b.at[slot], sem.at[slot, 0]
        ).wait()
        pltpu.make_async_copy(
            td_hbm.at[pl.ds(k * _DRB, _DRB), :], td_b.at[slot], sem.at[slot, 1]
        ).wait()
        pltpu.make_async_copy(
            od_hbm.at[pl.ds(k * _DRB, _DRB), :], od_b.at[slot], sem.at[slot, 2]
        ).wait()

    start(0, 0)

    def body(k, acc):
        slot = lax.rem(k, 2)
        nslot = lax.rem(k + 1, 2)

        @pl.when(k + 1 < _GRID)
        def _pf():
            start(k + 1, nslot)

        wait(k, slot)
        d = od_b[slot] - td_b[slot]
        ad = jnp.abs(d)
        m = jnp.minimum(ad, 1.0)
        f = m * (ad - 0.5 * m)
        g = jnp.sum(f.reshape(_RB, 4, 128), axis=1)
        pos = (ts_b[slot] > 0.0).astype(jnp.float32)
        return acc + jnp.sum(pos * g)

    out_ref[0, 0] = lax.fori_loop(0, _GRID, body, 0.0)


def _combine_body(sc_ref, reg_ref, o_ref):
    x = sc_ref[...]
    bce = jnp.sum(x[:, 0:16])
    nv = jnp.sum(x[:, 16:32])
    npos = jnp.sum(x[:, 32:48])
    reg = reg_ref[0, 0] + jnp.sum(x[:, 48:64])
    o_ref[0, 0] = bce / jnp.maximum(nv, 1.0) + 10.0 * reg / jnp.maximum(EPS, npos)


def kernel(target_deltas, target_scores, output_deltas, output_scores):
    ts = target_scores.reshape(N)
    osc = output_scores.reshape(N)
    ts2 = target_scores.reshape(N // 128, 128)
    td1 = target_deltas.reshape(N // 128, 128, 4).transpose(0, 2, 1).reshape(N * 4)
    od1 = output_deltas.reshape(N // 128, 128, 4).transpose(0, 2, 1).reshape(N * 4)
    td8 = td1.reshape(N // 32, 128)
    od8 = od1.reshape(N // 32, 128)

    reg_tc = pl.pallas_call(
        _tc_reg_body,
        in_specs=[
            pl.BlockSpec(memory_space=pltpu.MemorySpace.HBM),
            pl.BlockSpec(memory_space=pltpu.MemorySpace.HBM),
            pl.BlockSpec(memory_space=pltpu.MemorySpace.HBM),
        ],
        out_specs=pl.BlockSpec(memory_space=pltpu.SMEM),
        out_shape=jax.ShapeDtypeStruct((1, 1), jnp.float32),
        scratch_shapes=[
            pltpu.VMEM((2, _RB, 128), jnp.float32),
            pltpu.VMEM((2, _DRB, 128), jnp.float32),
            pltpu.VMEM((2, _DRB, 128), jnp.float32),
            pltpu.SemaphoreType.DMA((2, 3)),
        ],
    )(ts2, td8, od8)

    sc_partials = _sc_call(ts, osc, td1, od1)

    out = pl.pallas_call(
        _combine_body,
        out_shape=jax.ShapeDtypeStruct((1, 1), jnp.float32),
        in_specs=[
            pl.BlockSpec(memory_space=pltpu.VMEM),
            pl.BlockSpec(memory_space=pltpu.SMEM),
        ],
        out_specs=pl.BlockSpec(memory_space=pltpu.SMEM),
    )(sc_partials, reg_tc)
    return out[0, 0]


# speedup vs baseline: 1.0490x; 1.0490x over previous
import functools

import jax
import jax.numpy as jnp
from jax import lax
from jax.experimental import pallas as pl
from jax.experimental.pallas import tpu as pltpu
from jax.experimental.pallas import tpu_sc as plsc

N = 262144
EPS = 1e-7
NW = 32
PA = N // NW
ITERS = PA // 16

TC_EIGHTHS = 8

LN2 = 0.6931471805599453
SQRT2 = 1.4142135623730951
_LOG_COEF = (
    -5.4488729807735065e-06,
    0.9998871159844557,
    -0.49911010866955874,
    0.33800562352226765,
    -0.2740800450170098,
    0.17224595127722797,
)


def _ln(q):
    bits = lax.bitcast_convert_type(q, jnp.int32)
    e = (bits >> 23) - 127
    m = lax.bitcast_convert_type((bits & 0x007FFFFF) | 0x3F800000, jnp.float32)
    big = m > SQRT2
    m = jnp.where(big, m * 0.5, m)
    ef = e.astype(jnp.float32) + jnp.where(big, 1.0, 0.0)
    u = m - 1.0
    p = jnp.full_like(q, _LOG_COEF[5])
    for c in _LOG_COEF[4::-1]:
        p = p * u + c
    return p + ef * LN2


def _sc_cls(ts_hbm, os_hbm, out_hbm, ts_v, os_v, acc_v, s0, s1):
    wid = lax.axis_index("s") * 2 + lax.axis_index("c")
    abase = wid * PA

    c0 = pltpu.async_copy(ts_hbm.at[pl.ds(abase, PA)], ts_v, s0)
    c1 = pltpu.async_copy(os_hbm.at[pl.ds(abase, PA)], os_v, s1)
    c0.wait()
    c1.wait()

    def one(a0):
        t = ts_v[pl.ds(a0, 16)]
        p = os_v[pl.ds(a0, 16)]
        valid = t >= 0.0
        pos = t > 0.0
        q = jnp.where(pos, p, 1.0 - p)
        bce = -_ln(q)
        return (jnp.where(valid, bce, 0.0),
                jnp.where(valid, 1.0, 0.0),
                jnp.where(pos, 1.0, 0.0))

    def body(i, carry):
        acc_bce, acc_nv, acc_np = carry
        a0 = pl.multiple_of(i * 64, 64)
        b0, v0, p0 = one(a0)
        b1, v1, p1 = one(a0 + 16)
        b2, v2, p2 = one(a0 + 32)
        b3, v3, p3 = one(a0 + 48)
        return (acc_bce + ((b0 + b1) + (b2 + b3)),
                acc_nv + ((v0 + v1) + (v2 + v3)),
                acc_np + ((p0 + p1) + (p2 + p3)))

    z = jnp.zeros((16,), jnp.float32)
    acc_bce, acc_nv, acc_np = lax.fori_loop(0, ITERS // 4, body, (z, z, z))

    acc_v[pl.ds(0, 16)] = acc_bce
    acc_v[pl.ds(16, 16)] = acc_nv
    acc_v[pl.ds(32, 16)] = acc_np
    pltpu.sync_copy(acc_v, out_hbm.at[wid])


_sc_call = functools.partial(
    pl.kernel,
    out_type=jax.ShapeDtypeStruct((NW, 48), jnp.float32),
    mesh=plsc.VectorSubcoreMesh(core_axis_name="c", subcore_axis_name="s"),
    scratch_types=[
        pltpu.VMEM((PA,), jnp.float32),
        pltpu.VMEM((PA,), jnp.float32),
        pltpu.VMEM((48,), jnp.float32),
        pltpu.SemaphoreType.DMA,
        pltpu.SemaphoreType.DMA,
    ],
    compiler_params=pltpu.CompilerParams(needs_layout_passes=False),
)(_sc_cls)


_GRID = TC_EIGHTHS
_RB = (N // 128) // 8
_DRB = 4 * _RB


def _tc_reg_body(ts_hbm, td_hbm, od_hbm, out_ref, ts_b, td_b, od_b, sem):
    def start(k, slot):
        pltpu.make_async_copy(
            ts_hbm.at[pl.ds(k * _RB, _RB), :], ts_b.at[slot], sem.at[slot, 0]
        ).start()
        pltpu.make_async_copy(
            td_hbm.at[pl.ds(k * _DRB, _DRB), :], td_b.at[slot], sem.at[slot, 1]
        ).start()
        pltpu.make_async_copy(
            od_hbm.at[pl.ds(k * _DRB, _DRB), :], od_b.at[slot], sem.at[slot, 2]
        ).start()

    def wait(k, slot):
        pltpu.make_async_copy(
            ts_hbm.at[pl.ds(k * _RB, _RB), :], ts_b.at[slot], sem.at[slot, 0]
        ).wait()
        pltpu.make_async_copy(
            td_hbm.at[pl.ds(k * _DRB, _DRB), :], td_b.at[slot], sem.at[slot, 1]
        ).wait()
        pltpu.make_async_copy(
            od_hbm.at[pl.ds(k * _DRB, _DRB), :], od_b.at[slot], sem.at[slot, 2]
        ).wait()

    start(0, 0)

    def body(k, acc):
        slot = lax.rem(k, 2)
        nslot = lax.rem(k + 1, 2)

        @pl.when(k + 1 < _GRID)
        def _pf():
            start(k + 1, nslot)

        wait(k, slot)
        d = od_b[slot] - td_b[slot]
        ad = jnp.abs(d)
        m = jnp.minimum(ad, 1.0)
        f = m * (ad - 0.5 * m)
        g = jnp.sum(f.reshape(_RB, 4, 128), axis=1)
        pos = (ts_b[slot] > 0.0).astype(jnp.float32)
        return acc + jnp.sum(pos * g)

    out_ref[0, 0] = lax.fori_loop(0, _GRID, body, 0.0)


def _combine_body(sc_ref, reg_ref, o_ref):
    x = sc_ref[...]
    bce = jnp.sum(x[:, 0:16])
    nv = jnp.sum(x[:, 16:32])
    npos = jnp.sum(x[:, 32:48])
    reg = reg_ref[0, 0]
    o_ref[0, 0] = bce / jnp.maximum(nv, 1.0) + 10.0 * reg / jnp.maximum(EPS, npos)


def kernel(target_deltas, target_scores, output_deltas, output_scores):
    ts = target_scores.reshape(N)
    osc = output_scores.reshape(N)
    ts2 = target_scores.reshape(N // 128, 128)
    td8 = target_deltas.reshape(N // 128, 128, 4).transpose(0, 2, 1).reshape(N // 32, 128)
    od8 = output_deltas.reshape(N // 128, 128, 4).transpose(0, 2, 1).reshape(N // 32, 128)

    reg_tc = pl.pallas_call(
        _tc_reg_body,
        in_specs=[
            pl.BlockSpec(memory_space=pltpu.MemorySpace.HBM),
            pl.BlockSpec(memory_space=pltpu.MemorySpace.HBM),
            pl.BlockSpec(memory_space=pltpu.MemorySpace.HBM),
        ],
        out_specs=pl.BlockSpec(memory_space=pltpu.SMEM),
        out_shape=jax.ShapeDtypeStruct((1, 1), jnp.float32),
        scratch_shapes=[
            pltpu.VMEM((2, _RB, 128), jnp.float32),
            pltpu.VMEM((2, _DRB, 128), jnp.float32),
            pltpu.VMEM((2, _DRB, 128), jnp.float32),
            pltpu.SemaphoreType.DMA((2, 3)),
        ],
    )(ts2, td8, od8)

    sc_partials = _sc_call(ts, osc)

    out = pl.pallas_call(
        _combine_body,
        out_shape=jax.ShapeDtypeStruct((1, 1), jnp.float32),
        in_specs=[
            pl.BlockSpec(memory_space=pltpu.VMEM),
            pl.BlockSpec(memory_space=pltpu.SMEM),
        ],
        out_specs=pl.BlockSpec(memory_space=pltpu.SMEM),
    )(sc_partials, reg_tc)
    return out[0, 0]
